# Initial kernel scaffold; baseline (speedup 1.0000x reference)
import jax
import jax.numpy as jnp
from jax.experimental import pallas as pl


def kernel(x, edge_index, W_self, W_neigh, b):
    raise NotImplementedError("write your pallas kernel here")


# speedup vs baseline: 3.8936x; 3.8936x over previous
import jax
import jax.numpy as jnp
from jax import lax
from jax.experimental import pallas as pl
from jax.experimental.pallas import tpu as pltpu
from jax.experimental.pallas import tpu_sc as plsc

N = 10000
E = 160000
D = 256
H = 128
NS = 16
L = 16

EPT = E // NS
EC = 80
NCH = EPT // EC
RC = 80
NRCH = N // RC
RPT = -(-NRCH // NS)
NP = 10240


def _sc_body(xcat, src, dst, out, acc, degsh, idx_v, dst_v, rows_v,
             rbuf, degloc, dmrg, invbuf, sem):
    c = lax.axis_index("c")
    s = lax.axis_index("s")
    zvec = jnp.zeros((L,), dtype=jnp.float32)
    ones = jnp.ones((L,), dtype=jnp.float32)

    def zrow(r, carry):
        for j in range(H // L):
            rbuf[r, pl.ds(j * L, L)] = zvec
        return carry

    lax.fori_loop(0, RC, zrow, 0)

    def zdeg(i, carry):
        degloc[pl.ds(i * L, L)] = zvec
        return carry

    lax.fori_loop(0, N // L, zdeg, 0)

    for k in range(RPT):
        cid = k * NS + s

        @pl.when(cid < NRCH)
        def _():
            pltpu.sync_copy(rbuf, acc.at[pl.ds(cid * RC, RC)])

    plsc.subcore_barrier()

    base = s * EPT
    off = c * N

    def chunk(k, carry):
        eoff = base + k * EC
        pltpu.sync_copy(src.at[pl.ds(eoff, EC)], idx_v)
        pltpu.sync_copy(dst.at[pl.ds(eoff, EC)], dst_v)
        for j in range(EC // L):
            idx_v[pl.ds(j * L, L)] = idx_v[pl.ds(j * L, L)] + off
        pltpu.async_copy(xcat.at[idx_v], rows_v, sem).wait()
        pltpu.sync_copy(rows_v, acc.at[dst_v], add=True)
        for j in range(EC // L):
            iv = dst_v[pl.ds(j * L, L)]
            plsc.addupdate_scatter(degloc, [iv], ones)
        return carry

    lax.fori_loop(0, NCH, chunk, 0)

    pltpu.sync_copy(degloc, degsh.at[pl.ds(s * NP, N)])
    plsc.subcore_barrier()

    for k in range(RPT):
        cid = k * NS + s

        @pl.when(cid < NRCH)
        def _():
            row0 = cid * RC
            pltpu.sync_copy(acc.at[pl.ds(row0, RC)], rbuf)
            for t in range(NS):
                pltpu.sync_copy(degsh.at[pl.ds(t * NP + row0, RC)],
                                dmrg.at[pl.ds(t * RC, RC)])
            for j in range(RC // L):
                tot = dmrg[pl.ds(j * L, L)]
                for t in range(1, NS):
                    tot = tot + dmrg[pl.ds(t * RC + j * L, L)]
                invbuf[pl.ds(j * L, L)] = 1.0 / jnp.maximum(tot, 1.0)

            def norm(r, carry):
                scale = invbuf[pl.ds(r, L)][0]
                for j in range(H // L):
                    rbuf[r, pl.ds(j * L, L)] = rbuf[r, pl.ds(j * L, L)] * scale
                return carry

            lax.fori_loop(0, RC, norm, 0)
            pltpu.sync_copy(rbuf, out.at[pl.ds(off + row0, RC)])


_sc_agg = pl.kernel(
    _sc_body,
    out_type=jax.ShapeDtypeStruct((2 * N, H), jnp.float32),
    mesh=plsc.VectorSubcoreMesh(core_axis_name="c", subcore_axis_name="s"),
    compiler_params=pltpu.CompilerParams(needs_layout_passes=False),
    scratch_types=[
        pltpu.VMEM_SHARED((N, H), jnp.float32),
        pltpu.VMEM_SHARED((NS * NP,), jnp.float32),
        pltpu.VMEM((EC,), jnp.int32),
        pltpu.VMEM((EC,), jnp.int32),
        pltpu.VMEM((EC, H), jnp.float32),
        pltpu.VMEM((RC, H), jnp.float32),
        pltpu.VMEM((N,), jnp.float32),
        pltpu.VMEM((NS * RC,), jnp.float32),
        pltpu.VMEM((RC + L,), jnp.float32),
        pltpu.SemaphoreType.DMA,
    ],
)


BN = 2000


def _tc_body(x_ref, h0_ref, h1_ref, ws_ref, wn0_ref, wn1_ref, b_ref, o_ref):
    o_ref[...] = (
        jnp.dot(x_ref[...], ws_ref[...], preferred_element_type=jnp.float32)
        + jnp.dot(h0_ref[...], wn0_ref[...], preferred_element_type=jnp.float32)
        + jnp.dot(h1_ref[...], wn1_ref[...], preferred_element_type=jnp.float32)
        + b_ref[...]
    )


_tc_dense = pl.pallas_call(
    _tc_body,
    grid=(N // BN,),
    in_specs=[
        pl.BlockSpec((BN, D), lambda i: (i, 0)),
        pl.BlockSpec((BN, H), lambda i: (i, 0)),
        pl.BlockSpec((BN, H), lambda i: (i, 0)),
        pl.BlockSpec((D, D), lambda i: (0, 0)),
        pl.BlockSpec((H, D), lambda i: (0, 0)),
        pl.BlockSpec((H, D), lambda i: (0, 0)),
        pl.BlockSpec((1, D), lambda i: (0, 0)),
    ],
    out_specs=pl.BlockSpec((BN, D), lambda i: (i, 0)),
    out_shape=jax.ShapeDtypeStruct((N, D), jnp.float32),
)


def kernel(x, edge_index, W_self, W_neigh, b):
    src = edge_index[0].astype(jnp.int32)
    dst = edge_index[1].astype(jnp.int32)
    xcat = jnp.concatenate([x[:, :H], x[:, H:]], axis=0)
    h = _sc_agg(xcat, src, dst)
    return _tc_dense(x, h[:N], h[N:], W_self, W_neigh[:H], W_neigh[H:],
                     b.reshape(1, D))


# speedup vs baseline: 4.5265x; 1.1626x over previous
import jax
import jax.numpy as jnp
from jax import lax
from jax.experimental import pallas as pl
from jax.experimental.pallas import tpu as pltpu
from jax.experimental.pallas import tpu_sc as plsc

N = 10000
E = 160000
D = 256
H = 128
NS = 16
L = 16

EPT = E // NS
EC = 80
NCH = EPT // EC
RC = 80
NRCH = N // RC
RPT = -(-NRCH // NS)
NP = 10240


def _sc_body(xcat, src, dst, out, acc, degsh, idx0, idx1, dst0, dst1,
             rows0, rows1, degloc, dmrg, invbuf, sg0, sg1):
    c = lax.axis_index("c")
    s = lax.axis_index("s")
    zvec = jnp.zeros((L,), dtype=jnp.float32)
    ones = jnp.ones((L,), dtype=jnp.float32)
    idx_b = (idx0, idx1)
    dst_b = (dst0, dst1)
    rows_b = (rows0, rows1)
    sg_b = (sg0, sg1)

    def zrow(r, carry):
        for j in range(H // L):
            rows0[r, pl.ds(j * L, L)] = zvec
        return carry

    lax.fori_loop(0, RC, zrow, 0)

    def zdeg(i, carry):
        degloc[pl.ds(i * L, L)] = zvec
        return carry

    lax.fori_loop(0, N // L, zdeg, 0)

    for k in range(RPT):
        cid = k * NS + s

        @pl.when(cid < NRCH)
        def _():
            pltpu.sync_copy(rows0, acc.at[pl.ds(cid * RC, RC)])

    plsc.subcore_barrier()

    base = s * EPT
    off = c * N

    def load_idx(k, b):
        eoff = base + k * EC
        pltpu.sync_copy(src.at[pl.ds(eoff, EC)], idx_b[b])
        pltpu.sync_copy(dst.at[pl.ds(eoff, EC)], dst_b[b])
        for j in range(EC // L):
            idx_b[b][pl.ds(j * L, L)] = idx_b[b][pl.ds(j * L, L)] + off

    def start_gather(b):
        pltpu.async_copy(xcat.at[idx_b[b]], rows_b[b], sg_b[b])

    def wait_gather(b):
        pltpu.make_async_copy(xcat.at[idx_b[b]], rows_b[b], sg_b[b]).wait()

    load_idx(0, 0)
    start_gather(0)

    def body(k2, carry):
        for b in range(2):
            k = k2 * 2 + b

            @pl.when(k < NCH)
            def _():
                wait_gather(b)

                @pl.when(k + 1 < NCH)
                def _():
                    load_idx(k + 1, 1 - b)
                    start_gather(1 - b)

                pltpu.sync_copy(rows_b[b], acc.at[dst_b[b]], add=True)
                for j in range(EC // L):
                    iv = dst_b[b][pl.ds(j * L, L)]
                    plsc.addupdate_scatter(degloc, [iv], ones)

        return carry

    lax.fori_loop(0, (NCH + 1) // 2, body, 0)

    pltpu.sync_copy(degloc, degsh.at[pl.ds(s * NP, N)])
    plsc.subcore_barrier()

    for k in range(RPT):
        cid = k * NS + s

        @pl.when(cid < NRCH)
        def _():
            row0 = cid * RC
            pltpu.sync_copy(acc.at[pl.ds(row0, RC)], rows0)
            for t in range(NS):
                pltpu.sync_copy(degsh.at[pl.ds(t * NP + row0, RC)],
                                dmrg.at[pl.ds(t * RC, RC)])
            for j in range(RC // L):
                tot = dmrg[pl.ds(j * L, L)]
                for t in range(1, NS):
                    tot = tot + dmrg[pl.ds(t * RC + j * L, L)]
                invbuf[pl.ds(j * L, L)] = 1.0 / jnp.maximum(tot, 1.0)

            def norm(r, carry):
                scale = invbuf[pl.ds(r, L)][0]
                for j in range(H // L):
                    rows0[r, pl.ds(j * L, L)] = rows0[r, pl.ds(j * L, L)] * scale
                return carry

            lax.fori_loop(0, RC, norm, 0)
            pltpu.sync_copy(rows0, out.at[pl.ds(off + row0, RC)])


_sc_agg = pl.kernel(
    _sc_body,
    out_type=jax.ShapeDtypeStruct((2 * N, H), jnp.float32),
    mesh=plsc.VectorSubcoreMesh(core_axis_name="c", subcore_axis_name="s"),
    compiler_params=pltpu.CompilerParams(needs_layout_passes=False),
    scratch_types=[
        pltpu.VMEM_SHARED((N, H), jnp.float32),
        pltpu.VMEM_SHARED((NS * NP,), jnp.float32),
        pltpu.VMEM((EC,), jnp.int32),
        pltpu.VMEM((EC,), jnp.int32),
        pltpu.VMEM((EC,), jnp.int32),
        pltpu.VMEM((EC,), jnp.int32),
        pltpu.VMEM((EC, H), jnp.float32),
        pltpu.VMEM((EC, H), jnp.float32),
        pltpu.VMEM((N,), jnp.float32),
        pltpu.VMEM((NS * RC,), jnp.float32),
        pltpu.VMEM((RC + L,), jnp.float32),
        pltpu.SemaphoreType.DMA,
        pltpu.SemaphoreType.DMA,
    ],
)


BN = 2000


def _tc_body(x_ref, h0_ref, h1_ref, ws_ref, wn0_ref, wn1_ref, b_ref, o_ref):
    o_ref[...] = (
        jnp.dot(x_ref[...], ws_ref[...], preferred_element_type=jnp.float32)
        + jnp.dot(h0_ref[...], wn0_ref[...], preferred_element_type=jnp.float32)
        + jnp.dot(h1_ref[...], wn1_ref[...], preferred_element_type=jnp.float32)
        + b_ref[...]
    )


_tc_dense = pl.pallas_call(
    _tc_body,
    grid=(N // BN,),
    in_specs=[
        pl.BlockSpec((BN, D), lambda i: (i, 0)),
        pl.BlockSpec((BN, H), lambda i: (i, 0)),
        pl.BlockSpec((BN, H), lambda i: (i, 0)),
        pl.BlockSpec((D, D), lambda i: (0, 0)),
        pl.BlockSpec((H, D), lambda i: (0, 0)),
        pl.BlockSpec((H, D), lambda i: (0, 0)),
        pl.BlockSpec((1, D), lambda i: (0, 0)),
    ],
    out_specs=pl.BlockSpec((BN, D), lambda i: (i, 0)),
    out_shape=jax.ShapeDtypeStruct((N, D), jnp.float32),
)


def kernel(x, edge_index, W_self, W_neigh, b):
    src = edge_index[0].astype(jnp.int32)
    dst = edge_index[1].astype(jnp.int32)
    xcat = jnp.concatenate([x[:, :H], x[:, H:]], axis=0)
    h = _sc_agg(xcat, src, dst)
    return _tc_dense(x, h[:N], h[N:], W_self, W_neigh[:H], W_neigh[H:],
                     b.reshape(1, D))


# speedup vs baseline: 6.4132x; 1.4168x over previous
import jax
import jax.numpy as jnp
from jax import lax
from jax.experimental import pallas as pl
from jax.experimental.pallas import tpu as pltpu
from jax.experimental.pallas import tpu_sc as plsc

N = 10000
E = 160000
D = 256
H = 128
NS = 16
L = 16

EPT = E // NS
EC = 80
NCH = EPT // EC
RC = 80
NRCH = N // RC
RPT = -(-NRCH // NS)
NP = 10240


def _sc_body(xcat, src2, dst, out, acc, degsh, idx0, idx1, dst0, dst1,
             rows0, rows1, degloc, dmrg, invbuf, sg0, sg1, si0, si1):
    c = lax.axis_index("c")
    s = lax.axis_index("s")
    zvec = jnp.zeros((L,), dtype=jnp.float32)
    ones = jnp.ones((L,), dtype=jnp.float32)
    idx_b = (idx0, idx1)
    dst_b = (dst0, dst1)
    rows_b = (rows0, rows1)
    sg_b = (sg0, sg1)
    si_b = (si0, si1)

    def zrow(r, carry):
        for j in range(H // L):
            rows0[r, pl.ds(j * L, L)] = zvec
        return carry

    lax.fori_loop(0, RC, zrow, 0)

    def zdeg(i, carry):
        degloc[pl.ds(i * L, L)] = zvec
        return carry

    lax.fori_loop(0, N // L, zdeg, 0)

    for k in range(RPT):
        cid = k * NS + s

        @pl.when(cid < NRCH)
        def _():
            pltpu.sync_copy(rows0, acc.at[pl.ds(cid * RC, RC)])

    plsc.subcore_barrier()

    base = s * EPT
    off = c * N

    sbase = c * E + base

    def start_load_idx(k, b):
        eoff = k * EC
        pltpu.async_copy(src2.at[pl.ds(sbase + eoff, EC)], idx_b[b], si_b[b])
        pltpu.async_copy(dst.at[pl.ds(base + eoff, EC)], dst_b[b], si_b[b])

    def wait_load_idx(k, b):
        eoff = k * EC
        pltpu.make_async_copy(src2.at[pl.ds(sbase + eoff, EC)], idx_b[b],
                              si_b[b]).wait()
        pltpu.make_async_copy(dst.at[pl.ds(base + eoff, EC)], dst_b[b],
                              si_b[b]).wait()

    def start_gather(b):
        pltpu.async_copy(xcat.at[idx_b[b]], rows_b[b], sg_b[b])

    def wait_gather(b):
        pltpu.make_async_copy(xcat.at[idx_b[b]], rows_b[b], sg_b[b]).wait()

    start_load_idx(0, 0)
    wait_load_idx(0, 0)
    start_gather(0)
    start_load_idx(1, 1)

    def body(k2, carry):
        for b in range(2):
            k = k2 * 2 + b

            @pl.when(k < NCH)
            def _():
                wait_gather(b)

                @pl.when(k + 1 < NCH)
                def _():
                    wait_load_idx(k + 1, 1 - b)
                    start_gather(1 - b)

                pltpu.sync_copy(rows_b[b], acc.at[dst_b[b]], add=True)
                for j in range(EC // L):
                    iv = dst_b[b][pl.ds(j * L, L)]
                    plsc.addupdate_scatter(degloc, [iv], ones)

                @pl.when(k + 2 < NCH)
                def _():
                    start_load_idx(k + 2, b)

        return carry

    lax.fori_loop(0, (NCH + 1) // 2, body, 0)

    pltpu.sync_copy(degloc, degsh.at[pl.ds(s * NP, N)])
    plsc.subcore_barrier()

    for k in range(RPT):
        cid = k * NS + s

        @pl.when(cid < NRCH)
        def _():
            row0 = cid * RC
            pltpu.sync_copy(acc.at[pl.ds(row0, RC)], rows0)
            for t in range(NS):
                pltpu.sync_copy(degsh.at[pl.ds(t * NP + row0, RC)],
                                dmrg.at[pl.ds(t * RC, RC)])
            for j in range(RC // L):
                tot = dmrg[pl.ds(j * L, L)]
                for t in range(1, NS):
                    tot = tot + dmrg[pl.ds(t * RC + j * L, L)]
                invbuf[pl.ds(j * L, L)] = 1.0 / jnp.maximum(tot, 1.0)

            def norm(r, carry):
                scale = invbuf[pl.ds(r, L)][0]
                for j in range(H // L):
                    rows0[r, pl.ds(j * L, L)] = rows0[r, pl.ds(j * L, L)] * scale
                return carry

            lax.fori_loop(0, RC, norm, 0)
            pltpu.sync_copy(rows0, out.at[pl.ds(off + row0, RC)])


_sc_agg = pl.kernel(
    _sc_body,
    out_type=jax.ShapeDtypeStruct((2 * N, H), jnp.float32),
    mesh=plsc.VectorSubcoreMesh(core_axis_name="c", subcore_axis_name="s"),
    compiler_params=pltpu.CompilerParams(needs_layout_passes=False),
    scratch_types=[
        pltpu.VMEM_SHARED((N, H), jnp.float32),
        pltpu.VMEM_SHARED((NS * NP,), jnp.float32),
        pltpu.VMEM((EC,), jnp.int32),
        pltpu.VMEM((EC,), jnp.int32),
        pltpu.VMEM((EC,), jnp.int32),
        pltpu.VMEM((EC,), jnp.int32),
        pltpu.VMEM((EC, H), jnp.float32),
        pltpu.VMEM((EC, H), jnp.float32),
        pltpu.VMEM((N,), jnp.float32),
        pltpu.VMEM((NS * RC,), jnp.float32),
        pltpu.VMEM((RC + L,), jnp.float32),
        pltpu.SemaphoreType.DMA,
        pltpu.SemaphoreType.DMA,
        pltpu.SemaphoreType.DMA,
        pltpu.SemaphoreType.DMA,
    ],
)


BN = 2000


def _tc_body(x_ref, h0_ref, h1_ref, ws_ref, wn0_ref, wn1_ref, b_ref, o_ref):
    o_ref[...] = (
        jnp.dot(x_ref[...], ws_ref[...], preferred_element_type=jnp.float32)
        + jnp.dot(h0_ref[...], wn0_ref[...], preferred_element_type=jnp.float32)
        + jnp.dot(h1_ref[...], wn1_ref[...], preferred_element_type=jnp.float32)
        + b_ref[...]
    )


_tc_dense = pl.pallas_call(
    _tc_body,
    grid=(N // BN,),
    in_specs=[
        pl.BlockSpec((BN, D), lambda i: (i, 0)),
        pl.BlockSpec((BN, H), lambda i: (i, 0)),
        pl.BlockSpec((BN, H), lambda i: (i, 0)),
        pl.BlockSpec((D, D), lambda i: (0, 0)),
        pl.BlockSpec((H, D), lambda i: (0, 0)),
        pl.BlockSpec((H, D), lambda i: (0, 0)),
        pl.BlockSpec((1, D), lambda i: (0, 0)),
    ],
    out_specs=pl.BlockSpec((BN, D), lambda i: (i, 0)),
    out_shape=jax.ShapeDtypeStruct((N, D), jnp.float32),
)


def kernel(x, edge_index, W_self, W_neigh, b):
    src = edge_index[0].astype(jnp.int32)
    dst = edge_index[1].astype(jnp.int32)
    xcat = jnp.concatenate([x[:, :H], x[:, H:]], axis=0)
    src2 = jnp.concatenate([src, src + N])
    h = _sc_agg(xcat, src2, dst)
    return _tc_dense(x, h[:N], h[N:], W_self, W_neigh[:H], W_neigh[H:],
                     b.reshape(1, D))


# speedup vs baseline: 6.5037x; 1.0141x over previous
import jax
import jax.numpy as jnp
from jax import lax
from jax.experimental import pallas as pl
from jax.experimental.pallas import tpu as pltpu
from jax.experimental.pallas import tpu_sc as plsc

N = 10000
E = 160000
D = 256
H = 128
NS = 16
L = 16

EPT = E // NS
EC = 80
NCH = EPT // EC
RC = 80
NRCH = N // RC
RPT = -(-NRCH // NS)
NP = 10240


def _sc_body(xcat, src2, dst, out, acc, degsh, idx0, idx1, dst0, dst1,
             rows0, rows1, degloc, dmrg, invbuf, sg0, sg1, si0, si1):
    c = lax.axis_index("c")
    s = lax.axis_index("s")
    zvec = jnp.zeros((L,), dtype=jnp.float32)
    ones = jnp.ones((L,), dtype=jnp.float32)
    idx_b = (idx0, idx1)
    dst_b = (dst0, dst1)
    rows_b = (rows0, rows1)
    sg_b = (sg0, sg1)
    si_b = (si0, si1)

    def zrow(r, carry):
        for j in range(H // L):
            rows0[r, pl.ds(j * L, L)] = zvec
        return carry

    lax.fori_loop(0, RC, zrow, 0)

    def zdeg(i, carry):
        degloc[pl.ds(i * L, L)] = zvec
        return carry

    lax.fori_loop(0, N // L, zdeg, 0)

    for k in range(RPT):
        cid = k * NS + s

        @pl.when(cid < NRCH)
        def _():
            pltpu.sync_copy(rows0, acc.at[pl.ds(cid * RC, RC)])

    plsc.subcore_barrier()

    base = s * EPT
    off = c * N

    sbase = c * E + base

    def start_load_idx(k, b):
        eoff = k * EC
        pltpu.async_copy(src2.at[pl.ds(sbase + eoff, EC)], idx_b[b], si_b[b])
        pltpu.async_copy(dst.at[pl.ds(base + eoff, EC)], dst_b[b], si_b[b])

    def wait_load_idx(k, b):
        eoff = k * EC
        pltpu.make_async_copy(src2.at[pl.ds(sbase + eoff, EC)], idx_b[b],
                              si_b[b]).wait()
        pltpu.make_async_copy(dst.at[pl.ds(base + eoff, EC)], dst_b[b],
                              si_b[b]).wait()

    def start_gather(b):
        pltpu.async_copy(xcat.at[idx_b[b]], rows_b[b], sg_b[b])

    def wait_gather(b):
        pltpu.make_async_copy(xcat.at[idx_b[b]], rows_b[b], sg_b[b]).wait()

    start_load_idx(0, 0)
    wait_load_idx(0, 0)
    start_gather(0)
    start_load_idx(1, 1)

    def body(k2, carry):
        for b in range(2):
            k = k2 * 2 + b

            @pl.when(k < NCH)
            def _():
                wait_gather(b)

                @pl.when(k + 1 < NCH)
                def _():
                    wait_load_idx(k + 1, 1 - b)
                    start_gather(1 - b)

                pltpu.sync_copy(rows_b[b], acc.at[dst_b[b]], add=True)
                for j in range(EC // L):
                    iv = dst_b[b][pl.ds(j * L, L)]
                    plsc.addupdate_scatter(degloc, [iv], ones)

                @pl.when(k + 2 < NCH)
                def _():
                    start_load_idx(k + 2, b)

        return carry

    lax.fori_loop(0, (NCH + 1) // 2, body, 0)

    pltpu.sync_copy(degloc, degsh.at[pl.ds(s * NP, N)])
    plsc.subcore_barrier()

    for k in range(RPT):
        cid = k * NS + s

        @pl.when(cid < NRCH)
        def _():
            row0 = cid * RC
            pltpu.sync_copy(acc.at[pl.ds(row0, RC)], rows0)
            for t in range(NS):
                pltpu.sync_copy(degsh.at[pl.ds(t * NP + row0, RC)],
                                dmrg.at[pl.ds(t * RC, RC)])
            for j in range(RC // L):
                tot = dmrg[pl.ds(j * L, L)]
                for t in range(1, NS):
                    tot = tot + dmrg[pl.ds(t * RC + j * L, L)]
                invbuf[pl.ds(j * L, L)] = 1.0 / jnp.maximum(tot, 1.0)

            def norm(r, carry):
                scale = invbuf[pl.ds(r, L)][0]
                for j in range(H // L):
                    rows0[r, pl.ds(j * L, L)] = rows0[r, pl.ds(j * L, L)] * scale
                return carry

            lax.fori_loop(0, RC, norm, 0)
            pltpu.sync_copy(rows0, out.at[pl.ds(off + row0, RC)])


_sc_agg = pl.kernel(
    _sc_body,
    out_type=jax.ShapeDtypeStruct((2 * N, H), jnp.float32),
    mesh=plsc.VectorSubcoreMesh(core_axis_name="c", subcore_axis_name="s"),
    compiler_params=pltpu.CompilerParams(needs_layout_passes=False),
    scratch_types=[
        pltpu.VMEM_SHARED((N, H), jnp.float32),
        pltpu.VMEM_SHARED((NS * NP,), jnp.float32),
        pltpu.VMEM((EC,), jnp.int32),
        pltpu.VMEM((EC,), jnp.int32),
        pltpu.VMEM((EC,), jnp.int32),
        pltpu.VMEM((EC,), jnp.int32),
        pltpu.VMEM((EC, H), jnp.float32),
        pltpu.VMEM((EC, H), jnp.float32),
        pltpu.VMEM((N,), jnp.float32),
        pltpu.VMEM((NS * RC,), jnp.float32),
        pltpu.VMEM((RC + L,), jnp.float32),
        pltpu.SemaphoreType.DMA,
        pltpu.SemaphoreType.DMA,
        pltpu.SemaphoreType.DMA,
        pltpu.SemaphoreType.DMA,
    ],
)


BN = 2000


def _tc_self_body(x_ref, ws_ref, b_ref, o_ref):
    o_ref[...] = (
        jnp.dot(x_ref[...], ws_ref[...], preferred_element_type=jnp.float32)
        + b_ref[...]
    )


_tc_self = pl.pallas_call(
    _tc_self_body,
    grid=(N // BN,),
    in_specs=[
        pl.BlockSpec((BN, D), lambda i: (i, 0)),
        pl.BlockSpec((D, D), lambda i: (0, 0)),
        pl.BlockSpec((1, D), lambda i: (0, 0)),
    ],
    out_specs=pl.BlockSpec((BN, D), lambda i: (i, 0)),
    out_shape=jax.ShapeDtypeStruct((N, D), jnp.float32),
)


def _tc_neigh_body(z_ref, h0_ref, h1_ref, wn0_ref, wn1_ref, o_ref):
    o_ref[...] = (
        z_ref[...]
        + jnp.dot(h0_ref[...], wn0_ref[...], preferred_element_type=jnp.float32)
        + jnp.dot(h1_ref[...], wn1_ref[...], preferred_element_type=jnp.float32)
    )


_tc_neigh = pl.pallas_call(
    _tc_neigh_body,
    grid=(N // BN,),
    in_specs=[
        pl.BlockSpec((BN, D), lambda i: (i, 0)),
        pl.BlockSpec((BN, H), lambda i: (i, 0)),
        pl.BlockSpec((BN, H), lambda i: (i, 0)),
        pl.BlockSpec((H, D), lambda i: (0, 0)),
        pl.BlockSpec((H, D), lambda i: (0, 0)),
    ],
    out_specs=pl.BlockSpec((BN, D), lambda i: (i, 0)),
    out_shape=jax.ShapeDtypeStruct((N, D), jnp.float32),
)


def kernel(x, edge_index, W_self, W_neigh, b):
    src = edge_index[0].astype(jnp.int32)
    dst = edge_index[1].astype(jnp.int32)
    xview = x.reshape(2 * N, H)
    s2 = src * 2
    src2 = jnp.concatenate([s2, s2 + 1])
    h = _sc_agg(xview, src2, dst)
    z = _tc_self(x, W_self, b.reshape(1, D))
    return _tc_neigh(z, h[:N], h[N:], W_neigh[:H], W_neigh[H:])


# speedup vs baseline: 7.2205x; 1.1102x over previous
import jax
import jax.numpy as jnp
from jax import lax
from jax.experimental import pallas as pl
from jax.experimental.pallas import tpu as pltpu
from jax.experimental.pallas import tpu_sc as plsc

N = 10000
E = 160000
D = 256
H = 128
NS = 16
L = 16

EPT = E // NS
EC = 80
NCH = EPT // EC
RC = 80
NRCH = N // RC
RPT = -(-NRCH // NS)
NP = 10240


def _sc_body(xview, src2, dst, h0o, h1o, dego, acc, idx0, idx1, dst0, dst1,
             rows0, rows1, degloc, sg0, sg1, si0, si1):
    c = lax.axis_index("c")
    s = lax.axis_index("s")
    zvec = jnp.zeros((L,), dtype=jnp.float32)
    ones = jnp.ones((L,), dtype=jnp.float32)
    idx_b = (idx0, idx1)
    dst_b = (dst0, dst1)
    rows_b = (rows0, rows1)
    sg_b = (sg0, sg1)
    si_b = (si0, si1)

    def zrow(r, carry):
        for j in range(H // L):
            rows0[r, pl.ds(j * L, L)] = zvec
        return carry

    lax.fori_loop(0, RC, zrow, 0)

    def zdeg(i, carry):
        degloc[pl.ds(i * L, L)] = zvec
        return carry

    lax.fori_loop(0, N // L, zdeg, 0)

    for k in range(RPT):
        cid = k * NS + s

        @pl.when(cid < NRCH)
        def _():
            pltpu.sync_copy(rows0, acc.at[pl.ds(cid * RC, RC)])

    plsc.subcore_barrier()

    base = s * EPT

    sbase = c * E + base

    def start_load_idx(k, b):
        eoff = k * EC
        pltpu.async_copy(src2.at[pl.ds(sbase + eoff, EC)], idx_b[b], si_b[b])
        pltpu.async_copy(dst.at[pl.ds(base + eoff, EC)], dst_b[b], si_b[b])

    def wait_load_idx(k, b):
        eoff = k * EC
        pltpu.make_async_copy(src2.at[pl.ds(sbase + eoff, EC)], idx_b[b],
                              si_b[b]).wait()
        pltpu.make_async_copy(dst.at[pl.ds(base + eoff, EC)], dst_b[b],
                              si_b[b]).wait()

    def start_gather(b):
        pltpu.async_copy(xview.at[idx_b[b]], rows_b[b], sg_b[b])

    def wait_gather(b):
        pltpu.make_async_copy(xview.at[idx_b[b]], rows_b[b], sg_b[b]).wait()

    start_load_idx(0, 0)
    wait_load_idx(0, 0)
    start_gather(0)
    start_load_idx(1, 1)

    def body(k2, carry):
        for b in range(2):
            k = k2 * 2 + b

            @pl.when(k < NCH)
            def _():
                wait_gather(b)

                @pl.when(k + 1 < NCH)
                def _():
                    wait_load_idx(k + 1, 1 - b)
                    start_gather(1 - b)

                pltpu.sync_copy(rows_b[b], acc.at[dst_b[b]], add=True)

                @pl.when(c == 0)
                def _():
                    for j in range(EC // L):
                        iv = dst_b[b][pl.ds(j * L, L)]
                        plsc.addupdate_scatter(degloc, [iv], ones)

                @pl.when(k + 2 < NCH)
                def _():
                    start_load_idx(k + 2, b)

        return carry

    lax.fori_loop(0, (NCH + 1) // 2, body, 0)

    @pl.when(c == 0)
    def _():
        pltpu.sync_copy(degloc, dego.at[pl.ds(s * NP, N)])

    plsc.subcore_barrier()

    for k in range(RPT):
        cid = k * NS + s

        @pl.when(cid < NRCH)
        def _():
            row0 = cid * RC

            @pl.when(c == 0)
            def _():
                pltpu.sync_copy(acc.at[pl.ds(row0, RC)],
                                h0o.at[pl.ds(row0, RC)])

            @pl.when(c == 1)
            def _():
                pltpu.sync_copy(acc.at[pl.ds(row0, RC)],
                                h1o.at[pl.ds(row0, RC)])


_sc_agg = pl.kernel(
    _sc_body,
    out_type=(
        jax.ShapeDtypeStruct((N, H), jnp.float32),
        jax.ShapeDtypeStruct((N, H), jnp.float32),
        jax.ShapeDtypeStruct((NS * NP,), jnp.float32),
    ),
    mesh=plsc.VectorSubcoreMesh(core_axis_name="c", subcore_axis_name="s"),
    compiler_params=pltpu.CompilerParams(needs_layout_passes=False),
    scratch_types=[
        pltpu.VMEM_SHARED((N, H), jnp.float32),
        pltpu.VMEM((EC,), jnp.int32),
        pltpu.VMEM((EC,), jnp.int32),
        pltpu.VMEM((EC,), jnp.int32),
        pltpu.VMEM((EC,), jnp.int32),
        pltpu.VMEM((EC, H), jnp.float32),
        pltpu.VMEM((EC, H), jnp.float32),
        pltpu.VMEM((N,), jnp.float32),
        pltpu.SemaphoreType.DMA,
        pltpu.SemaphoreType.DMA,
        pltpu.SemaphoreType.DMA,
        pltpu.SemaphoreType.DMA,
    ],
)


BN = 2000


def _tc_self_body(x_ref, ws_ref, b_ref, o_ref):
    o_ref[...] = (
        jnp.dot(x_ref[...], ws_ref[...], preferred_element_type=jnp.float32)
        + b_ref[...]
    )


_tc_self = pl.pallas_call(
    _tc_self_body,
    grid=(N // BN,),
    in_specs=[
        pl.BlockSpec((BN, D), lambda i: (i, 0)),
        pl.BlockSpec((D, D), lambda i: (0, 0)),
        pl.BlockSpec((1, D), lambda i: (0, 0)),
    ],
    out_specs=pl.BlockSpec((BN, D), lambda i: (i, 0)),
    out_shape=jax.ShapeDtypeStruct((N, D), jnp.float32),
)


def _tc_neigh_body(z_ref, h0_ref, h1_ref, dg_ref, wn0_ref, wn1_ref, o_ref):
    deg = jnp.sum(dg_ref[...], axis=1)
    rdeg = (1.0 / jnp.maximum(deg, 1.0))[:, None]
    o_ref[...] = (
        z_ref[...]
        + jnp.dot(h0_ref[...] * rdeg, wn0_ref[...],
                  preferred_element_type=jnp.float32)
        + jnp.dot(h1_ref[...] * rdeg, wn1_ref[...],
                  preferred_element_type=jnp.float32)
    )


_tc_neigh = pl.pallas_call(
    _tc_neigh_body,
    grid=(N // BN,),
    in_specs=[
        pl.BlockSpec((BN, D), lambda i: (i, 0)),
        pl.BlockSpec((BN, H), lambda i: (i, 0)),
        pl.BlockSpec((BN, H), lambda i: (i, 0)),
        pl.BlockSpec((BN, NS), lambda i: (i, 0)),
        pl.BlockSpec((H, D), lambda i: (0, 0)),
        pl.BlockSpec((H, D), lambda i: (0, 0)),
    ],
    out_specs=pl.BlockSpec((BN, D), lambda i: (i, 0)),
    out_shape=jax.ShapeDtypeStruct((N, D), jnp.float32),
)


def kernel(x, edge_index, W_self, W_neigh, b):
    src = edge_index[0].astype(jnp.int32)
    dst = edge_index[1].astype(jnp.int32)
    xview = x.reshape(2 * N, H)
    s2 = src * 2
    src2 = jnp.concatenate([s2, s2 + 1])
    h0, h1, dego = _sc_agg(xview, src2, dst)
    z = _tc_self(x, W_self, b.reshape(1, D))
    deg16 = dego.reshape(NS, NP)[:, :N].T
    return _tc_neigh(z, h0, h1, deg16, W_neigh[:H], W_neigh[H:])


# speedup vs baseline: 7.2256x; 1.0007x over previous
import jax
import jax.numpy as jnp
from jax import lax
from jax.experimental import pallas as pl
from jax.experimental.pallas import tpu as pltpu
from jax.experimental.pallas import tpu_sc as plsc

N = 10000
E = 160000
D = 256
H = 128
NS = 16
L = 16

EPT = E // NS
EC = 80
NCH = EPT // EC
RC = 80
NRCH = N // RC
RPT = -(-NRCH // NS)
NP = 10240


def _sc_body(xview, src2, dst, h0o, h1o, dego, acc, idx0, idx1, dst0, dst1,
             rows0, rows1, degloc, sg0, sg1, si0, si1):
    c = lax.axis_index("c")
    s = lax.axis_index("s")
    zvec = jnp.zeros((L,), dtype=jnp.float32)
    ones = jnp.ones((L,), dtype=jnp.float32)
    idx_b = (idx0, idx1)
    dst_b = (dst0, dst1)
    rows_b = (rows0, rows1)
    sg_b = (sg0, sg1)
    si_b = (si0, si1)

    def zrow(r, carry):
        for j in range(H // L):
            rows0[r, pl.ds(j * L, L)] = zvec
        return carry

    lax.fori_loop(0, RC, zrow, 0)

    def zdeg(i, carry):
        degloc[pl.ds(i * L, L)] = zvec
        return carry

    lax.fori_loop(0, N // L, zdeg, 0)

    for k in range(RPT):
        cid = k * NS + s

        @pl.when(cid < NRCH)
        def _():
            pltpu.sync_copy(rows0, acc.at[pl.ds(cid * RC, RC)])

    plsc.subcore_barrier()

    base = s * EPT

    sbase = c * E + base

    def start_load_idx(k, b):
        eoff = k * EC
        pltpu.async_copy(src2.at[pl.ds(sbase + eoff, EC)], idx_b[b], si_b[b])
        pltpu.async_copy(dst.at[pl.ds(base + eoff, EC)], dst_b[b], si_b[b])

    def wait_load_idx(k, b):
        eoff = k * EC
        pltpu.make_async_copy(src2.at[pl.ds(sbase + eoff, EC)], idx_b[b],
                              si_b[b]).wait()
        pltpu.make_async_copy(dst.at[pl.ds(base + eoff, EC)], dst_b[b],
                              si_b[b]).wait()

    def start_gather(b):
        pltpu.async_copy(xview.at[idx_b[b]], rows_b[b], sg_b[b])

    def wait_gather(b):
        pltpu.make_async_copy(xview.at[idx_b[b]], rows_b[b], sg_b[b]).wait()

    start_load_idx(0, 0)
    wait_load_idx(0, 0)
    start_gather(0)
    start_load_idx(1, 1)

    def body(k2, carry):
        for b in range(2):
            k = k2 * 2 + b

            @pl.when(k < NCH)
            def _():
                wait_gather(b)

                @pl.when(k + 1 < NCH)
                def _():
                    wait_load_idx(k + 1, 1 - b)
                    start_gather(1 - b)

                pltpu.sync_copy(rows_b[b], acc.at[dst_b[b]], add=True)

                @pl.when(c == 0)
                def _():
                    for j in range(EC // L):
                        iv = dst_b[b][pl.ds(j * L, L)]
                        plsc.addupdate_scatter(degloc, [iv], ones)

                @pl.when(k + 2 < NCH)
                def _():
                    start_load_idx(k + 2, b)

        return carry

    lax.fori_loop(0, (NCH + 1) // 2, body, 0)

    @pl.when(c == 0)
    def _():
        pltpu.sync_copy(degloc, dego.at[pl.ds(s * NP, N)])

    plsc.subcore_barrier()

    for k in range(RPT):
        cid = k * NS + s

        @pl.when(cid < NRCH)
        def _():
            row0 = cid * RC

            @pl.when(c == 0)
            def _():
                pltpu.sync_copy(acc.at[pl.ds(row0, RC)],
                                h0o.at[pl.ds(row0, RC)])

            @pl.when(c == 1)
            def _():
                pltpu.sync_copy(acc.at[pl.ds(row0, RC)],
                                h1o.at[pl.ds(row0, RC)])


_sc_agg = pl.kernel(
    _sc_body,
    out_type=(
        jax.ShapeDtypeStruct((N, H), jnp.float32),
        jax.ShapeDtypeStruct((N, H), jnp.float32),
        jax.ShapeDtypeStruct((NS * NP,), jnp.float32),
    ),
    mesh=plsc.VectorSubcoreMesh(core_axis_name="c", subcore_axis_name="s"),
    compiler_params=pltpu.CompilerParams(needs_layout_passes=False),
    scratch_types=[
        pltpu.VMEM_SHARED((N, H), jnp.float32),
        pltpu.VMEM((EC,), jnp.int32),
        pltpu.VMEM((EC,), jnp.int32),
        pltpu.VMEM((EC,), jnp.int32),
        pltpu.VMEM((EC,), jnp.int32),
        pltpu.VMEM((EC, H), jnp.float32),
        pltpu.VMEM((EC, H), jnp.float32),
        pltpu.VMEM((N,), jnp.float32),
        pltpu.SemaphoreType.DMA,
        pltpu.SemaphoreType.DMA,
        pltpu.SemaphoreType.DMA,
        pltpu.SemaphoreType.DMA,
    ],
)


BN = 2000


def _tc_body(x_ref, h0_ref, h1_ref, dg_ref, ws_ref, wn0_ref, wn1_ref,
             b_ref, o_ref):
    deg = jnp.sum(dg_ref[...], axis=1)
    rdeg = (1.0 / jnp.maximum(deg, 1.0))[:, None]
    o_ref[...] = (
        jnp.dot(x_ref[...], ws_ref[...], preferred_element_type=jnp.float32)
        + jnp.dot(h0_ref[...] * rdeg, wn0_ref[...],
                  preferred_element_type=jnp.float32)
        + jnp.dot(h1_ref[...] * rdeg, wn1_ref[...],
                  preferred_element_type=jnp.float32)
        + b_ref[...]
    )


_tc_dense = pl.pallas_call(
    _tc_body,
    grid=(N // BN,),
    in_specs=[
        pl.BlockSpec((BN, D), lambda i: (i, 0)),
        pl.BlockSpec((BN, H), lambda i: (i, 0)),
        pl.BlockSpec((BN, H), lambda i: (i, 0)),
        pl.BlockSpec((BN, NS), lambda i: (i, 0)),
        pl.BlockSpec((D, D), lambda i: (0, 0)),
        pl.BlockSpec((H, D), lambda i: (0, 0)),
        pl.BlockSpec((H, D), lambda i: (0, 0)),
        pl.BlockSpec((1, D), lambda i: (0, 0)),
    ],
    out_specs=pl.BlockSpec((BN, D), lambda i: (i, 0)),
    out_shape=jax.ShapeDtypeStruct((N, D), jnp.float32),
)


def kernel(x, edge_index, W_self, W_neigh, b):
    src = edge_index[0].astype(jnp.int32)
    dst = edge_index[1].astype(jnp.int32)
    xview = x.reshape(2 * N, H)
    s2 = src * 2
    src2 = jnp.concatenate([s2, s2 + 1])
    h0, h1, dego = _sc_agg(xview, src2, dst)
    deg16 = dego.reshape(NS, NP)[:, :N].T
    return _tc_dense(x, h0, h1, deg16, W_self, W_neigh[:H], W_neigh[H:],
                     b.reshape(1, D))
